# baseline (speedup 1.0000x reference)
import functools

import jax
import jax.numpy as jnp
from jax import lax
from jax.experimental import pallas as pl
from jax.experimental.pallas import tpu as pltpu
from jax.experimental.pallas import tpu_sc as plsc

BATCH = 16384
SEQ = 200
D = 100
D2 = 2 * D
DQ = 208
VOCAB = 400
B = BATCH * SEQ
PAIRS = B // 2

NC = 2
NS = 16
NW = NC * NS

GP = 128
IDX_W = 2 * GP
N_CHUNKS = PAIRS // NW // GP
ROWS2 = B // IDX_W


def _sc_pair_gather(off2, q_tab):
    mesh = plsc.VectorSubcoreMesh(core_axis_name="c", subcore_axis_name="s")

    @functools.partial(
        pl.kernel,
        mesh=mesh,
        out_type=jax.ShapeDtypeStruct((PAIRS, D2), jnp.float32),
        scratch_types=[
            pltpu.VMEM((2, IDX_W), jnp.int32),
            pltpu.VMEM((2, GP), jnp.int32),
            pltpu.VMEM((2, GP, DQ), jnp.float32),
            pltpu.SemaphoreType.DMA,
            pltpu.SemaphoreType.DMA,
            pltpu.SemaphoreType.DMA,
            pltpu.SemaphoreType.DMA,
        ],
        compiler_params=pltpu.CompilerParams(
            use_tc_tiling_on_sc=False, needs_layout_passes=False),
    )
    def k(off_hbm, q_hbm, out_hbm, idx_v, pidx_v, rows_v, g0, g1, o0, o1):
        cid = lax.axis_index("c")
        sid = lax.axis_index("s")
        wid = sid * NC + cid
        chunk0 = wid * N_CHUNKS
        gsem = (g0, g1)
        osem = (o0, o1)

        def fire(i, b):
            pltpu.sync_copy(off_hbm.at[chunk0 + i], idx_v.at[b])
            for g in range(GP // 16):
                ivec = lax.iota(jnp.int32, 16) * 2 + (g * 32)
                i0 = plsc.load_gather(idx_v.at[b], [ivec])
                i1 = plsc.load_gather(idx_v.at[b], [ivec + 1])
                pidx_v[b, pl.ds(g * 16, 16)] = i0 * VOCAB + i1
            pltpu.async_copy(q_hbm.at[pidx_v.at[b]], rows_v.at[b], gsem[b])

        def drain_and_put(i, b):
            pltpu.make_async_copy(
                q_hbm.at[pidx_v.at[b]], rows_v.at[b], gsem[b]).wait()
            p0 = (chunk0 + i) * GP
            pltpu.async_copy(rows_v.at[b, :, pl.ds(0, D2)],
                             out_hbm.at[pl.ds(p0, GP)], osem[b])

        def wait_out(i, b):
            p0 = (chunk0 + i) * GP
            pltpu.make_async_copy(rows_v.at[b, :, pl.ds(0, D2)],
                                  out_hbm.at[pl.ds(p0, GP)], osem[b]).wait()

        def pair_step(p, carry):
            for b in range(2):
                i = p * 2 + b

                @pl.when(i >= 2)
                def _():
                    wait_out(i - 2, b)

                fire(i, b)

                @pl.when(i >= 1)
                def _():
                    drain_and_put(i - 1, 1 - b)

            return carry

        lax.fori_loop(0, N_CHUNKS // 2, pair_step, 0)
        drain_and_put(N_CHUNKS - 1, (N_CHUNKS - 1) % 2)
        wait_out(N_CHUNKS - 2, 0)
        wait_out(N_CHUNKS - 1, 1)

    return k(off2, q_tab)


def kernel(offsets, offset_embedding):
    off2 = offsets.reshape(ROWS2, IDX_W)
    left = jnp.broadcast_to(
        offset_embedding[:, None, :], (VOCAB, VOCAB, D))
    right_pad = jnp.pad(offset_embedding, ((0, 0), (0, DQ - D2)))
    right = jnp.broadcast_to(
        right_pad[None, :, :], (VOCAB, VOCAB, DQ - D))
    q_tab = jnp.concatenate([left, right], axis=-1).reshape(VOCAB * VOCAB, DQ)
    out = _sc_pair_gather(off2, q_tab)
    return out.reshape(BATCH, SEQ, D)


# speedup vs baseline: 1.0376x; 1.0376x over previous
import functools

import jax
import jax.numpy as jnp
from jax import lax
from jax.experimental import pallas as pl
from jax.experimental.pallas import tpu as pltpu
from jax.experimental.pallas import tpu_sc as plsc

BATCH = 16384
SEQ = 200
D = 100
D2 = 2 * D
VOCAB = 400
B = BATCH * SEQ
PAIRS = B // 2

NC = 2
NS = 16
NW = NC * NS
REP = NW

GP = 128
IDX_W = 2 * GP
N_CHUNKS = PAIRS // NW // GP
ROWS2 = B // IDX_W

WA = 96
WF = 8
WFP = 16
WB = 96


def _sc_pair_gather(off2, tab_a, tab_b, tab_f):
    mesh = plsc.VectorSubcoreMesh(core_axis_name="c", subcore_axis_name="s")

    @functools.partial(
        pl.kernel,
        mesh=mesh,
        out_type=jax.ShapeDtypeStruct((PAIRS, D2), jnp.float32),
        scratch_types=[
            pltpu.VMEM((2, IDX_W), jnp.int32),
            pltpu.VMEM((2, GP), jnp.int32),
            pltpu.VMEM((2, GP), jnp.int32),
            pltpu.VMEM((2, GP), jnp.int32),
            pltpu.VMEM((2, GP, WA), jnp.float32),
            pltpu.VMEM((2, GP, WB), jnp.float32),
            pltpu.VMEM((2, GP, WFP), jnp.float32),
            pltpu.SemaphoreType.DMA,
            pltpu.SemaphoreType.DMA,
            pltpu.SemaphoreType.DMA,
            pltpu.SemaphoreType.DMA,
        ],
        compiler_params=pltpu.CompilerParams(
            use_tc_tiling_on_sc=False, needs_layout_passes=False),
    )
    def k(off_hbm, a_hbm, b_hbm, f_hbm, out_hbm,
          idx_v, i0_v, i1_v, pidx_v, a_v, b_v, f_v, g0, g1, o0, o1):
        cid = lax.axis_index("c")
        sid = lax.axis_index("s")
        wid = sid * NC + cid
        chunk0 = wid * N_CHUNKS
        rbase = wid * VOCAB
        gsem = (g0, g1)
        osem = (o0, o1)

        def fire(i, b):
            pltpu.sync_copy(off_hbm.at[chunk0 + i], idx_v.at[b])
            for g in range(GP // 16):
                ivec = lax.iota(jnp.int32, 16) * 2 + (g * 32)
                i0 = plsc.load_gather(idx_v.at[b], [ivec])
                i1 = plsc.load_gather(idx_v.at[b], [ivec + 1])
                i0_v[b, pl.ds(g * 16, 16)] = i0 + rbase
                i1_v[b, pl.ds(g * 16, 16)] = i1 + rbase
                pidx_v[b, pl.ds(g * 16, 16)] = i0 * VOCAB + i1
            pltpu.async_copy(a_hbm.at[i0_v.at[b]], a_v.at[b], gsem[b])
            pltpu.async_copy(b_hbm.at[i1_v.at[b]], b_v.at[b], gsem[b])
            pltpu.async_copy(f_hbm.at[pidx_v.at[b]], f_v.at[b], gsem[b])

        def drain(i, b):
            pltpu.make_async_copy(a_hbm.at[i0_v.at[b]], a_v.at[b],
                                  gsem[b]).wait()
            pltpu.make_async_copy(b_hbm.at[i1_v.at[b]], b_v.at[b],
                                  gsem[b]).wait()
            pltpu.make_async_copy(f_hbm.at[pidx_v.at[b]], f_v.at[b],
                                  gsem[b]).wait()

        def put(i, b):
            p0 = (chunk0 + i) * GP
            rows = out_hbm.at[pl.ds(p0, GP)]
            pltpu.async_copy(a_v.at[b], rows.at[:, pl.ds(0, WA)], osem[b])
            pltpu.async_copy(f_v.at[b, :, pl.ds(0, WF)],
                             rows.at[:, pl.ds(WA, WF)], osem[b])
            pltpu.async_copy(b_v.at[b], rows.at[:, pl.ds(WA + WF, WB)],
                             osem[b])

        def wait_out(i, b):
            p0 = (chunk0 + i) * GP
            rows = out_hbm.at[pl.ds(p0, GP)]
            pltpu.make_async_copy(a_v.at[b], rows.at[:, pl.ds(0, WA)],
                                  osem[b]).wait()
            pltpu.make_async_copy(f_v.at[b, :, pl.ds(0, WF)],
                                  rows.at[:, pl.ds(WA, WF)], osem[b]).wait()
            pltpu.make_async_copy(b_v.at[b], rows.at[:, pl.ds(WA + WF, WB)],
                                  osem[b]).wait()

        def pair_step(p, carry):
            for b in range(2):
                i = p * 2 + b

                @pl.when(i >= 2)
                def _():
                    wait_out(i - 2, b)

                fire(i, b)

                @pl.when(i >= 1)
                def _():
                    drain(i - 1, 1 - b)
                    put(i - 1, 1 - b)

            return carry

        lax.fori_loop(0, N_CHUNKS // 2, pair_step, 0)
        drain(N_CHUNKS - 1, (N_CHUNKS - 1) % 2)
        put(N_CHUNKS - 1, (N_CHUNKS - 1) % 2)
        wait_out(N_CHUNKS - 2, 0)
        wait_out(N_CHUNKS - 1, 1)

    return k(off2, tab_a, tab_b, tab_f)


def kernel(offsets, offset_embedding):
    off2 = offsets.reshape(ROWS2, IDX_W)
    tab_a = jnp.broadcast_to(
        offset_embedding[None, :, :WA], (REP, VOCAB, WA)).reshape(-1, WA)
    tab_b = jnp.broadcast_to(
        offset_embedding[None, :, D - WB:], (REP, VOCAB, WB)).reshape(-1, WB)
    tails = jnp.broadcast_to(
        offset_embedding[:, None, WA:D], (VOCAB, VOCAB, D - WA))
    heads = jnp.broadcast_to(
        offset_embedding[None, :, :WF - (D - WA)],
        (VOCAB, VOCAB, WF - (D - WA)))
    pad = jnp.zeros((VOCAB, VOCAB, WFP - WF), jnp.float32)
    tab_f = jnp.concatenate(
        [tails, heads, pad], axis=-1).reshape(VOCAB * VOCAB, WFP)
    out = _sc_pair_gather(off2, tab_a, tab_b, tab_f)
    return out.reshape(BATCH, SEQ, D)


# speedup vs baseline: 2.9094x; 2.8039x over previous
import functools

import jax
import jax.numpy as jnp
from jax import lax
from jax.experimental import pallas as pl
from jax.experimental.pallas import tpu as pltpu
from jax.experimental.pallas import tpu_sc as plsc

BATCH = 16384
SEQ = 200
D = 100
DP = 112
VOCAB = 400
B = BATCH * SEQ

NC = 2
NS = 16
NW = NC * NS

G = 128
K = 2
ROWS = B // G
ROWS_PER_W = ROWS // NW
N_CHUNKS = ROWS_PER_W // K


def _sc_gather(off2, tab_padded):
    mesh = plsc.VectorSubcoreMesh(core_axis_name="c", subcore_axis_name="s")

    @functools.partial(
        pl.kernel,
        mesh=mesh,
        out_type=jax.ShapeDtypeStruct((ROWS, G, DP), jnp.float32),
        scratch_types=[
            pltpu.VMEM_SHARED((VOCAB, DP), jnp.float32),
            pltpu.VMEM((2, K, G), jnp.int32),
            pltpu.VMEM((2, K, G, DP), jnp.float32),
            pltpu.SemaphoreType.DMA,
            pltpu.SemaphoreType.DMA,
            pltpu.SemaphoreType.DMA,
            pltpu.SemaphoreType.DMA,
        ],
    )
    def k(off_hbm, tab_hbm, out_hbm, tab_s, idx_v, rows_v, g0, g1, o0, o1):
        cid = lax.axis_index("c")
        sid = lax.axis_index("s")
        wid = sid * NC + cid
        row0 = wid * ROWS_PER_W
        gsem = (g0, g1)
        osem = (o0, o1)

        @pl.when(sid == 0)
        def _():
            pltpu.sync_copy(tab_hbm, tab_s)

        plsc.subcore_barrier()

        def fire(i, b):
            r = row0 + i * K
            pltpu.sync_copy(off_hbm.at[pl.ds(r, K)], idx_v.at[b])
            for j in range(K):
                pltpu.async_copy(
                    tab_s.at[idx_v.at[b].at[j]], rows_v.at[b].at[j], gsem[b])

        def drain_and_put(i, b):
            for j in range(K):
                pltpu.make_async_copy(
                    tab_s.at[idx_v.at[b].at[j]], rows_v.at[b].at[j],
                    gsem[b]).wait()
            r = row0 + i * K
            pltpu.async_copy(rows_v.at[b], out_hbm.at[pl.ds(r, K)], osem[b])

        def wait_out(i, b):
            r = row0 + i * K
            pltpu.make_async_copy(
                rows_v.at[b], out_hbm.at[pl.ds(r, K)], osem[b]).wait()

        def pair(p, carry):
            for b in range(2):
                i = p * 2 + b

                @pl.when(i >= 2)
                def _():
                    wait_out(i - 2, b)

                fire(i, b)

                @pl.when(i >= 1)
                def _():
                    drain_and_put(i - 1, 1 - b)

            return carry

        lax.fori_loop(0, N_CHUNKS // 2, pair, 0)
        drain_and_put(N_CHUNKS - 1, (N_CHUNKS - 1) % 2)
        wait_out(N_CHUNKS - 2, 0)
        wait_out(N_CHUNKS - 1, 1)

    return k(off2, tab_padded)


def kernel(offsets, offset_embedding):
    off2 = offsets.reshape(ROWS, G)
    tab_padded = jnp.pad(offset_embedding, ((0, 0), (0, DP - D)))
    out = _sc_gather(off2, tab_padded)
    return out[:, :, :D].reshape(BATCH, SEQ, D)
